# baseline (speedup 1.0000x reference)
import functools

import jax
import jax.numpy as jnp
from jax import lax
from jax.experimental import pallas as pl
from jax.experimental.pallas import tpu as pltpu
from jax.experimental.pallas import tpu_sc as plsc

_INFO = plsc.get_sparse_core_info()
_NC = _INFO.num_cores
_NS = _INFO.num_subcores
_NW = _NC * _NS
_L = 16
_CHUNK = 128


def _make_phase1(batch, n_rows, tail_len):
    seg = (n_rows // (_NS * _CHUNK)) * _CHUNK
    half = seg // 2
    tail_start = n_rows - tail_len
    b_per_t = batch // _NS
    n_chunks = b_per_t // _CHUNK
    mesh = plsc.VectorSubcoreMesh(core_axis_name="c", subcore_axis_name="s")

    @functools.partial(
        pl.kernel,
        out_type=jax.ShapeDtypeStruct((4 * batch,), jnp.float32),
        mesh=mesh,
        scratch_types=[
            pltpu.VMEM_SHARED((n_rows,), jnp.float32),
            pltpu.VMEM((half,), jnp.float32),
            pltpu.VMEM((half,), jnp.float32),
            pltpu.VMEM((tail_len,), jnp.float32),
            pltpu.VMEM((n_chunks, _CHUNK), jnp.int32),
            pltpu.VMEM((b_per_t,), jnp.float32),
            pltpu.VMEM((b_per_t,), jnp.float32),
            pltpu.SemaphoreType.DMA,
            pltpu.SemaphoreType.DMA,
            pltpu.SemaphoreType.DMA,
            pltpu.SemaphoreType.DMA,
            pltpu.SemaphoreType.DMA,
            pltpu.SemaphoreType.DMA,
        ],
    )
    def phase1(user_hbm, item_hbm, ut_hbm, it_hbm, utt_hbm, itt_hbm, out_hbm,
               sp, va, vb, vt, idx_v, g0, g1,
               sem_ha, sem_hb, sem_sa, sem_sb, sem_g, sem_w):
        cid = lax.axis_index("c")
        sid = lax.axis_index("s")
        base = sid * b_per_t
        start = sid * seg

        def read_half(row, buf, off, sem):
            @pl.when(cid == 0)
            def _():
                pltpu.async_copy(ut_hbm.at[row, pl.ds(start + off, half)],
                                 buf, sem)

            @pl.when(cid == 1)
            def _():
                pltpu.async_copy(it_hbm.at[row, pl.ds(start + off, half)],
                                 buf, sem)

            return pltpu.make_async_copy(
                ut_hbm.at[row, pl.ds(start + off, half)], buf, sem)

        @pl.when(cid == 0)
        def _():
            for k in range(n_chunks):
                pltpu.sync_copy(
                    user_hbm.at[pl.ds(base + k * _CHUNK, _CHUNK)],
                    idx_v.at[k])

        @pl.when(cid == 1)
        def _():
            for k in range(n_chunks):
                pltpu.sync_copy(
                    item_hbm.at[pl.ds(base + k * _CHUNK, _CHUNK)],
                    idx_v.at[k])

        ra = read_half(0, va, 0, sem_ha)
        rb = read_half(0, vb, half, sem_hb)
        ra.wait()
        sa = pltpu.async_copy(va, sp.at[pl.ds(start, half)], sem_sa)
        rb.wait()
        sb = pltpu.async_copy(vb, sp.at[pl.ds(start + half, half)], sem_sb)
        sa.wait()
        ra1 = read_half(1, va, 0, sem_ha)
        sb.wait()
        rb1 = read_half(1, vb, half, sem_hb)

        @pl.when(jnp.logical_and(sid == 0, cid == 0))
        def _():
            pltpu.sync_copy(utt_hbm.at[0, pl.ds(0, tail_len)], vt)

        @pl.when(jnp.logical_and(sid == 0, cid == 1))
        def _():
            pltpu.sync_copy(itt_hbm.at[0, pl.ds(0, tail_len)], vt)

        @pl.when(sid == 0)
        def _():
            pltpu.sync_copy(vt, sp.at[pl.ds(tail_start, tail_len)])

        plsc.subcore_barrier()

        g0_copies = [pltpu.async_copy(sp.at[idx_v.at[k]],
                                      g0.at[pl.ds(k * _CHUNK, _CHUNK)], sem_g)
                     for k in range(n_chunks)]
        for c in g0_copies:
            c.wait()
        w0 = pltpu.async_copy(
            g0, out_hbm.at[pl.ds(2 * cid * batch + sid * b_per_t, b_per_t)],
            sem_w)
        plsc.subcore_barrier()

        ra1.wait()
        sa1 = pltpu.async_copy(va, sp.at[pl.ds(start, half)], sem_sa)
        rb1.wait()
        sb1 = pltpu.async_copy(vb, sp.at[pl.ds(start + half, half)], sem_sb)
        sa1.wait()
        sb1.wait()

        @pl.when(jnp.logical_and(sid == 0, cid == 0))
        def _():
            pltpu.sync_copy(utt_hbm.at[1, pl.ds(0, tail_len)], vt)

        @pl.when(jnp.logical_and(sid == 0, cid == 1))
        def _():
            pltpu.sync_copy(itt_hbm.at[1, pl.ds(0, tail_len)], vt)

        @pl.when(sid == 0)
        def _():
            pltpu.sync_copy(vt, sp.at[pl.ds(tail_start, tail_len)])

        plsc.subcore_barrier()

        g1_copies = [pltpu.async_copy(sp.at[idx_v.at[k]],
                                      g1.at[pl.ds(k * _CHUNK, _CHUNK)], sem_g)
                     for k in range(n_chunks)]
        for c in g1_copies:
            c.wait()
        pltpu.sync_copy(
            g1, out_hbm.at[pl.ds((2 * cid + 1) * batch + sid * b_per_t,
                                 b_per_t)])
        w0.wait()

    return phase1

def _make_phase2(rows, cols):
    def body(cols4, o):
        o[...] = (cols4[0] * cols4[2] + cols4[1] * cols4[3])

    return pl.pallas_call(
        body,
        out_shape=jax.ShapeDtypeStruct((rows, cols), jnp.float32),
    )


def kernel(user, item, user_table, item_table):
    batch = user.shape[0]
    n_rows = user_table.shape[0]
    tail_len = 5 * _CHUNK
    p1 = _make_phase1(batch, n_rows, tail_len)
    rows = batch // 128
    p2 = _make_phase2(rows, 128)
    cols = p1(user.astype(jnp.int32), item.astype(jnp.int32),
              user_table.T, item_table.T,
              user_table[n_rows - tail_len:].T,
              item_table[n_rows - tail_len:].T)
    out2d = p2(cols.reshape(4, rows, 128))
    return out2d.reshape(batch)


# speedup vs baseline: 1.0953x; 1.0953x over previous
import functools

import jax
import jax.numpy as jnp
from jax import lax
from jax.experimental import pallas as pl
from jax.experimental.pallas import tpu as pltpu
from jax.experimental.pallas import tpu_sc as plsc

_INFO = plsc.get_sparse_core_info()
_NC = _INFO.num_cores
_NS = _INFO.num_subcores
_NW = _NC * _NS
_L = 16
_CHUNK = 128


def _make_phase1(batch, n_rows, tail_len):
    seg = (n_rows // (_NS * _CHUNK)) * _CHUNK
    half = seg // 2
    tail_start = n_rows - tail_len
    b_per_t = batch // _NS
    n_chunks = b_per_t // _CHUNK
    mesh = plsc.VectorSubcoreMesh(core_axis_name="c", subcore_axis_name="s")

    @functools.partial(
        pl.kernel,
        out_type=jax.ShapeDtypeStruct((4 * batch,), jnp.float32),
        mesh=mesh,
        scratch_types=[
            pltpu.VMEM_SHARED((n_rows,), jnp.float32),
            pltpu.VMEM((half,), jnp.float32),
            pltpu.VMEM((half,), jnp.float32),
            pltpu.VMEM((tail_len,), jnp.float32),
            pltpu.VMEM((b_per_t,), jnp.int32),
            pltpu.VMEM((b_per_t,), jnp.float32),
            pltpu.VMEM((b_per_t,), jnp.float32),
            pltpu.SemaphoreType.DMA,
            pltpu.SemaphoreType.DMA,
            pltpu.SemaphoreType.DMA,
            pltpu.SemaphoreType.DMA,
            pltpu.SemaphoreType.DMA,
            pltpu.SemaphoreType.DMA,
        ],
    )
    def phase1(user_hbm, item_hbm, ut_hbm, it_hbm, utt_hbm, itt_hbm, out_hbm,
               sp, va, vb, vt, idx_v, g0, g1,
               sem_ha, sem_hb, sem_sa, sem_sb, sem_g, sem_w):
        cid = lax.axis_index("c")
        sid = lax.axis_index("s")
        base = sid * b_per_t
        start = sid * seg

        def read_half(row, buf, off, sem):
            @pl.when(cid == 0)
            def _():
                pltpu.async_copy(ut_hbm.at[row, pl.ds(start + off, half)],
                                 buf, sem)

            @pl.when(cid == 1)
            def _():
                pltpu.async_copy(it_hbm.at[row, pl.ds(start + off, half)],
                                 buf, sem)

            return pltpu.make_async_copy(
                ut_hbm.at[row, pl.ds(start + off, half)], buf, sem)

        @pl.when(cid == 0)
        def _():
            pltpu.sync_copy(user_hbm.at[pl.ds(base, b_per_t)], idx_v)

        @pl.when(cid == 1)
        def _():
            pltpu.sync_copy(item_hbm.at[pl.ds(base, b_per_t)], idx_v)

        ra = read_half(0, va, 0, sem_ha)
        rb = read_half(0, vb, half, sem_hb)
        ra.wait()
        sa = pltpu.async_copy(va, sp.at[pl.ds(start, half)], sem_sa)
        rb.wait()
        sb = pltpu.async_copy(vb, sp.at[pl.ds(start + half, half)], sem_sb)
        sa.wait()
        ra1 = read_half(1, va, 0, sem_ha)
        sb.wait()
        rb1 = read_half(1, vb, half, sem_hb)

        @pl.when(jnp.logical_and(sid == 0, cid == 0))
        def _():
            pltpu.sync_copy(utt_hbm.at[0, pl.ds(0, tail_len)], vt)

        @pl.when(jnp.logical_and(sid == 0, cid == 1))
        def _():
            pltpu.sync_copy(itt_hbm.at[0, pl.ds(0, tail_len)], vt)

        @pl.when(sid == 0)
        def _():
            pltpu.sync_copy(vt, sp.at[pl.ds(tail_start, tail_len)])

        plsc.subcore_barrier()

        pltpu.async_copy(sp.at[idx_v], g0, sem_g).wait()
        w0 = pltpu.async_copy(
            g0, out_hbm.at[pl.ds(2 * cid * batch + sid * b_per_t, b_per_t)],
            sem_w)
        plsc.subcore_barrier()

        ra1.wait()
        sa1 = pltpu.async_copy(va, sp.at[pl.ds(start, half)], sem_sa)
        rb1.wait()
        sb1 = pltpu.async_copy(vb, sp.at[pl.ds(start + half, half)], sem_sb)
        sa1.wait()
        sb1.wait()

        @pl.when(jnp.logical_and(sid == 0, cid == 0))
        def _():
            pltpu.sync_copy(utt_hbm.at[1, pl.ds(0, tail_len)], vt)

        @pl.when(jnp.logical_and(sid == 0, cid == 1))
        def _():
            pltpu.sync_copy(itt_hbm.at[1, pl.ds(0, tail_len)], vt)

        @pl.when(sid == 0)
        def _():
            pltpu.sync_copy(vt, sp.at[pl.ds(tail_start, tail_len)])

        plsc.subcore_barrier()

        pltpu.async_copy(sp.at[idx_v], g1, sem_g).wait()
        pltpu.sync_copy(
            g1, out_hbm.at[pl.ds((2 * cid + 1) * batch + sid * b_per_t,
                                 b_per_t)])
        w0.wait()

    return phase1

def _make_phase2(rows, cols):
    def body(cols4, o):
        o[...] = (cols4[0] * cols4[2] + cols4[1] * cols4[3])

    return pl.pallas_call(
        body,
        out_shape=jax.ShapeDtypeStruct((rows, cols), jnp.float32),
    )


def kernel(user, item, user_table, item_table):
    batch = user.shape[0]
    n_rows = user_table.shape[0]
    tail_len = 5 * _CHUNK
    p1 = _make_phase1(batch, n_rows, tail_len)
    rows = batch // 128
    p2 = _make_phase2(rows, 128)
    cols = p1(user.astype(jnp.int32), item.astype(jnp.int32),
              user_table.T, item_table.T,
              user_table[n_rows - tail_len:].T,
              item_table[n_rows - tail_len:].T)
    out2d = p2(cols.reshape(4, rows, 128))
    return out2d.reshape(batch)


# speedup vs baseline: 1.1067x; 1.0104x over previous
import functools

import jax
import jax.numpy as jnp
from jax import lax
from jax.experimental import pallas as pl
from jax.experimental.pallas import tpu as pltpu
from jax.experimental.pallas import tpu_sc as plsc

_INFO = plsc.get_sparse_core_info()
_NC = _INFO.num_cores
_NS = _INFO.num_subcores
_NW = _NC * _NS
_L = 16
_CHUNK = 128


def _make_phase1(batch, n_rows, tail_len):
    seg = (n_rows // (_NS * _CHUNK)) * _CHUNK
    half = seg // 2
    tail_start = n_rows - tail_len
    b_per_t = batch // _NS
    n_chunks = b_per_t // _CHUNK
    mesh = plsc.VectorSubcoreMesh(core_axis_name="c", subcore_axis_name="s")

    @functools.partial(
        pl.kernel,
        out_type=jax.ShapeDtypeStruct((4 * batch,), jnp.float32),
        mesh=mesh,
        scratch_types=[
            pltpu.VMEM_SHARED((n_rows,), jnp.float32),
            pltpu.VMEM((half,), jnp.float32),
            pltpu.VMEM((half,), jnp.float32),
            pltpu.VMEM((tail_len,), jnp.float32),
            pltpu.VMEM((tail_len,), jnp.float32),
            pltpu.VMEM((b_per_t,), jnp.int32),
            pltpu.VMEM((b_per_t,), jnp.float32),
            pltpu.VMEM((b_per_t,), jnp.float32),
            pltpu.SemaphoreType.DMA,
            pltpu.SemaphoreType.DMA,
            pltpu.SemaphoreType.DMA,
            pltpu.SemaphoreType.DMA,
            pltpu.SemaphoreType.DMA,
            pltpu.SemaphoreType.DMA,
        ],
    )
    def phase1(user_hbm, item_hbm, ut_hbm, it_hbm, utt_hbm, itt_hbm, out_hbm,
               sp, va, vb, vt0, vt1, idx_v, g0, g1,
               sem_ha, sem_hb, sem_sa, sem_sb, sem_g, sem_w):
        cid = lax.axis_index("c")
        sid = lax.axis_index("s")
        base = sid * b_per_t
        start = sid * seg

        def read_half(row, buf, off, sem):
            @pl.when(cid == 0)
            def _():
                pltpu.async_copy(ut_hbm.at[row, pl.ds(start + off, half)],
                                 buf, sem)

            @pl.when(cid == 1)
            def _():
                pltpu.async_copy(it_hbm.at[row, pl.ds(start + off, half)],
                                 buf, sem)

            return pltpu.make_async_copy(
                ut_hbm.at[row, pl.ds(start + off, half)], buf, sem)

        @pl.when(cid == 0)
        def _():
            pltpu.sync_copy(user_hbm.at[pl.ds(base, b_per_t)], idx_v)

        @pl.when(cid == 1)
        def _():
            pltpu.sync_copy(item_hbm.at[pl.ds(base, b_per_t)], idx_v)

        @pl.when(jnp.logical_and(sid == 0, cid == 0))
        def _():
            pltpu.async_copy(utt_hbm.at[0, pl.ds(0, tail_len)], vt0, sem_w)
            pltpu.async_copy(utt_hbm.at[1, pl.ds(0, tail_len)], vt1, sem_w)

        @pl.when(jnp.logical_and(sid == 0, cid == 1))
        def _():
            pltpu.async_copy(itt_hbm.at[0, pl.ds(0, tail_len)], vt0, sem_w)
            pltpu.async_copy(itt_hbm.at[1, pl.ds(0, tail_len)], vt1, sem_w)

        ra = read_half(0, va, 0, sem_ha)
        rb = read_half(0, vb, half, sem_hb)
        ra.wait()
        sa = pltpu.async_copy(va, sp.at[pl.ds(start, half)], sem_sa)
        rb.wait()
        sb = pltpu.async_copy(vb, sp.at[pl.ds(start + half, half)], sem_sb)
        sa.wait()
        ra1 = read_half(1, va, 0, sem_ha)
        sb.wait()
        rb1 = read_half(1, vb, half, sem_hb)

        @pl.when(sid == 0)
        def _():
            pltpu.make_async_copy(utt_hbm.at[0, pl.ds(0, tail_len)],
                                  vt0, sem_w).wait()
            pltpu.make_async_copy(utt_hbm.at[1, pl.ds(0, tail_len)],
                                  vt1, sem_w).wait()
            pltpu.sync_copy(vt0, sp.at[pl.ds(tail_start, tail_len)])

        plsc.subcore_barrier()

        pltpu.async_copy(sp.at[idx_v], g0, sem_g).wait()
        w0 = pltpu.async_copy(
            g0, out_hbm.at[pl.ds(2 * cid * batch + sid * b_per_t, b_per_t)],
            sem_w)
        plsc.subcore_barrier()

        ra1.wait()
        sa1 = pltpu.async_copy(va, sp.at[pl.ds(start, half)], sem_sa)
        rb1.wait()
        sb1 = pltpu.async_copy(vb, sp.at[pl.ds(start + half, half)], sem_sb)
        sa1.wait()
        sb1.wait()

        @pl.when(sid == 0)
        def _():
            pltpu.sync_copy(vt1, sp.at[pl.ds(tail_start, tail_len)])

        plsc.subcore_barrier()

        pltpu.async_copy(sp.at[idx_v], g1, sem_g).wait()
        pltpu.sync_copy(
            g1, out_hbm.at[pl.ds((2 * cid + 1) * batch + sid * b_per_t,
                                 b_per_t)])
        w0.wait()

    return phase1

def _make_phase2(rows, cols):
    def body(cols4, o):
        o[...] = (cols4[0] * cols4[2] + cols4[1] * cols4[3])

    return pl.pallas_call(
        body,
        out_shape=jax.ShapeDtypeStruct((rows, cols), jnp.float32),
    )


def kernel(user, item, user_table, item_table):
    batch = user.shape[0]
    n_rows = user_table.shape[0]
    tail_len = 5 * _CHUNK
    p1 = _make_phase1(batch, n_rows, tail_len)
    rows = batch // 128
    p2 = _make_phase2(rows, 128)
    cols = p1(user.astype(jnp.int32), item.astype(jnp.int32),
              user_table.T, item_table.T,
              user_table[n_rows - tail_len:].T,
              item_table[n_rows - tail_len:].T)
    out2d = p2(cols.reshape(4, rows, 128))
    return out2d.reshape(batch)
